# baseline (speedup 1.0000x reference)
import functools
import math

import jax
import jax.numpy as jnp
from jax import lax
from jax.experimental import pallas as pl
from jax.experimental.pallas import tpu as pltpu
from jax.experimental.pallas import tpu_sc as plsc

S = 64.0
MARGIN = 0.5
COS_M = math.cos(MARGIN)
SIN_M = math.sin(MARGIN)

N_ROWS = 1024
N_COLS = 100000
EDGE = (N_COLS // 128) * 128
CMAX = EDGE - 128

BR = 8


def _sqrt16(y):
    yb = lax.bitcast_convert_type(y, jnp.int32)
    seed = lax.bitcast_convert_type(jnp.int32(0x5F3759DF) - (yb >> 1), jnp.float32)
    t = seed
    for _ in range(3):
        t = t * (1.5 - 0.5 * y * t * t)
    return y * t


def _sc_adjust(logits, labels):
    info = plsc.get_sparse_core_info()
    nw = info.num_cores * info.num_subcores
    per_w = N_ROWS // nw
    mesh = plsc.VectorSubcoreMesh(core_axis_name="c", subcore_axis_name="s")

    @functools.partial(
        pl.kernel,
        out_type=jax.ShapeDtypeStruct((N_ROWS,), jnp.float32),
        mesh=mesh,
        scratch_types=[
            pltpu.VMEM((per_w,), jnp.int32),
            pltpu.VMEM((per_w * 8, 128), jnp.float32),
            pltpu.VMEM((per_w,), jnp.float32),
            pltpu.SemaphoreType.DMA,
        ],
    )
    def sc_kernel(logits_hbm, labels_hbm, adj_hbm, lab_v, stage_v, adj_v, sem):
        wid = lax.axis_index("s") * info.num_cores + lax.axis_index("c")
        base = pl.multiple_of(wid * per_w, per_w)
        lane = lax.iota(jnp.int32, 16)
        pltpu.sync_copy(labels_hbm.at[pl.ds(base, per_w)], lab_v)
        copies = []
        for k in range(per_w // 16):
            lab16 = lab_v[pl.ds(k * 16, 16)]
            safe = jnp.maximum(lab16, 0)
            c128 = jnp.minimum(safe & ~jnp.int32(127), CMAX)
            for r in range(16):
                i = k * 16 + r
                c_s = pl.multiple_of(c128[r], 128)
                row0 = base + (i // 8) * 8
                copies.append(
                    pltpu.async_copy(
                        logits_hbm.at[pl.ds(row0, 8), pl.ds(c_s, 128)],
                        stage_v.at[pl.ds(i * 8, 8), :],
                        sem,
                    )
                )
        for c in copies:
            c.wait()
        for k in range(per_w // 16):
            lab16k = lab_v[pl.ds(k * 16, 16)]
            safek = jnp.maximum(lab16k, 0)
            c128k = jnp.minimum(safek & ~jnp.int32(127), CMAX)
            wvec = jnp.minimum(safek - c128k, 127)
            x = jnp.zeros((16,), jnp.float32) + 0.0
            for r in range(16):
                i = k * 16 + r
                w = wvec[r]
                vec16 = stage_v[i * 8 + (i % 8), pl.ds(w & ~jnp.int32(15), 16)]
                sp = lax.gather(
                    vec16,
                    jnp.full((16, 1), w & 15, jnp.int32),
                    lax.GatherDimensionNumbers(
                        offset_dims=(), collapsed_slice_dims=(0,),
                        start_index_map=(0,)),
                    slice_sizes=(1,),
                    mode=lax.GatherScatterMode.PROMISE_IN_BOUNDS,
                )
                x = jnp.where(lane == r, sp, x)
            lab16 = lab_v[pl.ds(k * 16, 16)]
            y = jnp.maximum(1.0 - x * x, 1e-30)
            adj = (x * COS_M - _sqrt16(y) * SIN_M) * S
            adj_v[pl.ds(k * 16, 16)] = jnp.where(lab16 >= 0, adj, x * S)
        pltpu.sync_copy(adj_v, adj_hbm.at[pl.ds(base, per_w)])

    return sc_kernel(logits, labels)


def _tc_block(labels_ref, adj_ref, x_ref, o_ref):
    i = pl.program_id(0)
    o_ref[...] = x_ref[...] * S
    for r in range(BR):
        c = labels_ref[i * BR + r]
        val = adj_ref[i * BR + r]

        @pl.when(c >= 0)
        def _():
            c128 = pl.multiple_of((c // 128) * 128, 128)
            row = o_ref[pl.ds(r, 1), pl.ds(c128, 128)]
            sel = lax.broadcasted_iota(jnp.int32, (1, 128), 1) == c - c128
            xrow = x_ref[pl.ds(r, 1), pl.ds(c128, 128)]
            y = jnp.maximum(1.0 - xrow * xrow, 0.0)
            adjv = (xrow * COS_M - jnp.sqrt(y) * SIN_M) * S
            v128 = jnp.where(c >= EDGE, adjv, jnp.full((1, 128), val, jnp.float32))
            o_ref[pl.ds(r, 1), pl.ds(c128, 128)] = jnp.where(sel, v128, row)


def kernel(logits, labels):
    n_rows, n_cols = logits.shape
    adj = _sc_adjust(logits, labels)
    return pl.pallas_call(
        _tc_block,
        grid=(n_rows // BR,),
        in_specs=[
            pl.BlockSpec(memory_space=pltpu.SMEM),
            pl.BlockSpec(memory_space=pltpu.SMEM),
            pl.BlockSpec((BR, n_cols), lambda i: (i, 0)),
        ],
        out_specs=pl.BlockSpec((BR, n_cols), lambda i: (i, 0)),
        out_shape=jax.ShapeDtypeStruct((n_rows, n_cols), logits.dtype),
    )(labels, adj, logits)


# speedup vs baseline: 1.0075x; 1.0075x over previous
import functools
import math

import jax
import jax.numpy as jnp
from jax import lax
from jax.experimental import pallas as pl
from jax.experimental.pallas import tpu as pltpu
from jax.experimental.pallas import tpu_sc as plsc

S = 64.0
MARGIN = 0.5
COS_M = math.cos(MARGIN)
SIN_M = math.sin(MARGIN)

N_ROWS = 1024
N_COLS = 100000
EDGE = (N_COLS // 128) * 128
CMAX = EDGE - 128

BR = 32


def _sqrt16(y):
    yb = lax.bitcast_convert_type(y, jnp.int32)
    seed = lax.bitcast_convert_type(jnp.int32(0x5F3759DF) - (yb >> 1), jnp.float32)
    t = seed
    for _ in range(3):
        t = t * (1.5 - 0.5 * y * t * t)
    return y * t


def _sc_adjust(logits, labels):
    info = plsc.get_sparse_core_info()
    nw = info.num_cores * info.num_subcores
    per_w = N_ROWS // nw
    mesh = plsc.VectorSubcoreMesh(core_axis_name="c", subcore_axis_name="s")

    @functools.partial(
        pl.kernel,
        out_type=jax.ShapeDtypeStruct((N_ROWS,), jnp.float32),
        mesh=mesh,
        scratch_types=[
            pltpu.VMEM((per_w,), jnp.int32),
            pltpu.VMEM((per_w * 8, 128), jnp.float32),
            pltpu.VMEM((per_w,), jnp.float32),
            pltpu.SemaphoreType.DMA,
        ],
    )
    def sc_kernel(logits_hbm, labels_hbm, adj_hbm, lab_v, stage_v, adj_v, sem):
        wid = lax.axis_index("s") * info.num_cores + lax.axis_index("c")
        base = pl.multiple_of(wid * per_w, per_w)
        lane = lax.iota(jnp.int32, 16)
        pltpu.sync_copy(labels_hbm.at[pl.ds(base, per_w)], lab_v)
        copies = []
        for k in range(per_w // 16):
            lab16 = lab_v[pl.ds(k * 16, 16)]
            safe = jnp.maximum(lab16, 0)
            c128 = jnp.minimum(safe & ~jnp.int32(127), CMAX)
            for r in range(16):
                i = k * 16 + r
                c_s = pl.multiple_of(c128[r], 128)
                row0 = base + (i // 8) * 8
                copies.append(
                    pltpu.async_copy(
                        logits_hbm.at[pl.ds(row0, 8), pl.ds(c_s, 128)],
                        stage_v.at[pl.ds(i * 8, 8), :],
                        sem,
                    )
                )
        for c in copies:
            c.wait()
        for k in range(per_w // 16):
            lab16k = lab_v[pl.ds(k * 16, 16)]
            safek = jnp.maximum(lab16k, 0)
            c128k = jnp.minimum(safek & ~jnp.int32(127), CMAX)
            wvec = jnp.minimum(safek - c128k, 127)
            x = jnp.zeros((16,), jnp.float32) + 0.0
            for r in range(16):
                i = k * 16 + r
                w = wvec[r]
                vec16 = stage_v[i * 8 + (i % 8), pl.ds(w & ~jnp.int32(15), 16)]
                sp = lax.gather(
                    vec16,
                    jnp.full((16, 1), w & 15, jnp.int32),
                    lax.GatherDimensionNumbers(
                        offset_dims=(), collapsed_slice_dims=(0,),
                        start_index_map=(0,)),
                    slice_sizes=(1,),
                    mode=lax.GatherScatterMode.PROMISE_IN_BOUNDS,
                )
                x = jnp.where(lane == r, sp, x)
            lab16 = lab_v[pl.ds(k * 16, 16)]
            y = jnp.maximum(1.0 - x * x, 1e-30)
            adj = (x * COS_M - _sqrt16(y) * SIN_M) * S
            adj_v[pl.ds(k * 16, 16)] = jnp.where(lab16 >= 0, adj, x * S)
        pltpu.sync_copy(adj_v, adj_hbm.at[pl.ds(base, per_w)])

    return sc_kernel(logits, labels)


TAIL = N_COLS - EDGE


def _tc_block(labels_ref, adj_ref, x_ref, o_ref):
    i = pl.program_id(0)
    o_ref[...] = x_ref[...] * S
    for r in range(BR):
        c = labels_ref[i * BR + r]
        val = adj_ref[i * BR + r]

        @pl.when((c >= 0) & (c < EDGE))
        def _():
            c128 = pl.multiple_of((c // 128) * 128, 128)
            row = o_ref[pl.ds(r, 1), pl.ds(c128, 128)]
            sel = lax.broadcasted_iota(jnp.int32, (1, 128), 1) == c - c128
            o_ref[pl.ds(r, 1), pl.ds(c128, 128)] = jnp.where(sel, val, row)

        @pl.when(c >= EDGE)
        def _():
            tail = x_ref[pl.ds(r, 1), pl.ds(EDGE, TAIL)]
            y = jnp.maximum(1.0 - tail * tail, 0.0)
            adjv = (tail * COS_M - jnp.sqrt(y) * SIN_M) * S
            sel = lax.broadcasted_iota(jnp.int32, (1, TAIL), 1) == c - EDGE
            o_ref[pl.ds(r, 1), pl.ds(EDGE, TAIL)] = jnp.where(sel, adjv, tail * S)


def kernel(logits, labels):
    n_rows, n_cols = logits.shape
    adj = _sc_adjust(logits, labels)
    return pl.pallas_call(
        _tc_block,
        grid=(n_rows // BR,),
        in_specs=[
            pl.BlockSpec(memory_space=pltpu.SMEM),
            pl.BlockSpec(memory_space=pltpu.SMEM),
            pl.BlockSpec((BR, n_cols), lambda i: (i, 0)),
        ],
        out_specs=pl.BlockSpec((BR, n_cols), lambda i: (i, 0)),
        out_shape=jax.ShapeDtypeStruct((n_rows, n_cols), logits.dtype),
    )(labels, adj, logits)
